# baseline (speedup 1.0000x reference)
import functools

import jax
import jax.numpy as jnp
from jax import lax
from jax.experimental import pallas as pl
from jax.experimental.pallas import tpu as pltpu
from jax.experimental.pallas import tpu_sc as plsc

N = 10000
D = 128
E = 320000

NC = 2
NS = 16
NW = NC * NS
EPW = E // NW
K = 80
NB = EPW // K
ROWS_PT = (N // NS) // 8 * 8
TAIL = N - NS * ROWS_PT
QS = 2048.0


def _proj_body(x_ref, w_ref, att_ref, h_ref, a2_ref):
    h = jnp.dot(x_ref[...], w_ref[...], preferred_element_type=jnp.float32)
    h_ref[...] = h
    a2_ref[...] = jnp.dot(h, att_ref[...], preferred_element_type=jnp.float32)


def _combine_body(p0_ref, p1_ref, h_ref, spt_ref, a2_ref, b_ref, o_ref):
    asum = a2_ref[:, 0] + a2_ref[:, 1]
    wself = jnp.exp(jnp.maximum(asum, 0.2 * asum))
    s = jnp.sum(spt_ref[...], axis=1) + wself + 1e-16
    num = p0_ref[...] + p1_ref[...] + h_ref[...] * wself[:, None]
    o_ref[...] = num / s[:, None] + b_ref[...]


def _sc_body(h_hbm, ei_hbm, api_hbm,
             outp_hbm, sp_hbm,
             eb0, eb1, dst0, dst1, api_v, s_v, w_v, rows0, rows1,
             acc_sh, gsem0, gsem1, ssem0, ssem1, esem0, esem1):
    cid = lax.axis_index("c")
    sid = lax.axis_index("s")
    wid = cid * NS + sid

    pltpu.sync_copy(api_hbm, api_v)

    zeros = jnp.zeros((16,), jnp.float32)

    @plsc.parallel_loop(0, N // 16, unroll=8)
    def _z_s(i):
        s_v[pl.ds(i * 16, 16)] = zeros

    @plsc.parallel_loop(0, K, unroll=4)
    def _z_rows(e):
        for j in range(D // 16):
            rows0[e, pl.ds(j * 16, 16)] = zeros

    base = sid * ROWS_PT
    for r in range(0, ROWS_PT, K):
        n = min(K, ROWS_PT - r)
        pltpu.async_copy(rows0.at[pl.ds(0, n)], acc_sh.at[pl.ds(base + r, n)],
                         gsem0)
    for r in range(0, ROWS_PT, K):
        n = min(K, ROWS_PT - r)
        pltpu.make_async_copy(rows0.at[pl.ds(0, n)],
                              acc_sh.at[pl.ds(base + r, n)], gsem0).wait()

    @pl.when(sid == 0)
    def _zero_tail():
        pltpu.sync_copy(rows0.at[pl.ds(0, TAIL)],
                        acc_sh.at[pl.ds(NS * ROWS_PT, TAIL)])
    plsc.subcore_barrier()

    def _weights(eb):
        for j in range(K // 16):
            sidx = eb[0, pl.ds(j * 16, 16)]
            didx = eb[1, pl.ds(j * 16, 16)]
            sw = plsc.load_gather(api_v, [sidx])
            dw = plsc.load_gather(api_v, [didx])
            av = (sw >> 16).astype(jnp.float32) * (1.0 / QS)
            bv = ((dw << 16) >> 16).astype(jnp.float32) * (1.0 / QS)
            x = av + bv
            wv = jnp.exp(jnp.maximum(x, 0.2 * x))
            w_v[pl.ds(j * 16, 16)] = wv
            plsc.addupdate_scatter(s_v, [didx], wv)

    def _scale_rows(rows):
        @plsc.parallel_loop(0, K, unroll=8)
        def _scale(e):
            wb = plsc.load_gather(w_v, [jnp.full((16,), 0, jnp.int32) + e])
            for j in range(D // 16):
                sl = pl.ds(j * 16, 16)
                rows[e, sl] = rows[e, sl] * wb

    def _snap_dst(eb, dst):
        for j in range(K // 16):
            dst[pl.ds(j * 16, 16)] = eb[1, pl.ds(j * 16, 16)]

    pltpu.sync_copy(ei_hbm.at[wid, 0], eb0)
    pltpu.async_copy(ei_hbm.at[wid, 1], eb1, esem1)
    pltpu.async_copy(h_hbm.at[eb0.at[0]], rows0, gsem0)

    def _half(b, eb_c, rows_c, dst_c, gsem_c, ssem_c, esem_c,
              eb_o, rows_o, dst_o, gsem_o, ssem_o, esem_o,
              prev_pred, eb_pred):
        _weights(eb_c)
        _snap_dst(eb_c, dst_c)
        if prev_pred is None:
            pltpu.make_async_copy(rows_o, acc_sh.at[dst_o], ssem_o).wait()
        else:
            @pl.when(prev_pred)
            def _wait_prev():
                pltpu.make_async_copy(rows_o, acc_sh.at[dst_o], ssem_o).wait()
        pltpu.make_async_copy(ei_hbm.at[wid, b + 1], eb_o, esem_o).wait()
        pltpu.async_copy(h_hbm.at[eb_o.at[0]], rows_o, gsem_o)
        if eb_pred is None:
            pltpu.async_copy(ei_hbm.at[wid, b + 2], eb_c, esem_c)
        else:
            @pl.when(eb_pred)
            def _next_eb():
                pltpu.async_copy(ei_hbm.at[wid, b + 2], eb_c, esem_c)
        pltpu.make_async_copy(h_hbm.at[eb_c.at[0]], rows_c, gsem_c).wait()
        _scale_rows(rows_c)
        pltpu.async_copy(rows_c, acc_sh.at[dst_c], ssem_c, add=True)

    def _pair(g, carry):
        b0 = 2 * g
        _half(b0, eb0, rows0, dst0, gsem0, ssem0, esem0,
              eb1, rows1, dst1, gsem1, ssem1, esem1, g >= 1, None)
        _half(b0 + 1, eb1, rows1, dst1, gsem1, ssem1, esem1,
              eb0, rows0, dst0, gsem0, ssem0, esem0, None,
              g < (NB - 1) // 2 - 1)
        return carry

    lax.fori_loop(0, (NB - 1) // 2, _pair, 0)

    pltpu.make_async_copy(h_hbm.at[eb0.at[0]], rows0, gsem0).wait()
    _weights(eb0)
    _snap_dst(eb0, dst0)
    pltpu.make_async_copy(rows1, acc_sh.at[dst1], ssem1).wait()
    _scale_rows(rows0)
    pltpu.async_copy(rows0, acc_sh.at[dst0], ssem0, add=True)
    pltpu.make_async_copy(rows0, acc_sh.at[dst0], ssem0).wait()

    pltpu.sync_copy(s_v, sp_hbm.at[wid])

    plsc.subcore_barrier()
    pltpu.sync_copy(acc_sh.at[pl.ds(base, ROWS_PT)],
                    outp_hbm.at[cid, pl.ds(base, ROWS_PT)])

    @pl.when(sid == 0)
    def _copy_tail():
        pltpu.sync_copy(acc_sh.at[pl.ds(NS * ROWS_PT, TAIL)],
                        outp_hbm.at[cid, pl.ds(NS * ROWS_PT, TAIL)])


def kernel(X, edge_index, W, att_src, att_dst, bias):
    att2 = jnp.stack([att_src, att_dst], axis=1)

    grid = (10,)
    bn = N // grid[0]
    h, a2 = pl.pallas_call(
        _proj_body,
        grid=grid,
        in_specs=[
            pl.BlockSpec((bn, D), lambda i: (i, 0)),
            pl.BlockSpec((D, D), lambda i: (0, 0)),
            pl.BlockSpec((D, 2), lambda i: (0, 0)),
        ],
        out_specs=[
            pl.BlockSpec((bn, D), lambda i: (i, 0)),
            pl.BlockSpec((bn, 2), lambda i: (i, 0)),
        ],
        out_shape=[
            jax.ShapeDtypeStruct((N, D), jnp.float32),
            jax.ShapeDtypeStruct((N, 2), jnp.float32),
        ],
    )(X, W, att2)

    aq = jnp.clip(jnp.round(a2 * QS), -32767.0, 32767.0).astype(jnp.int32)
    api = (aq[:, 0] << 16) | (aq[:, 1] & 0xFFFF)

    ei3 = edge_index.reshape(2, NW, NB, K).transpose(1, 2, 0, 3)

    mesh = plsc.VectorSubcoreMesh(core_axis_name="c", subcore_axis_name="s")
    sc = functools.partial(
        pl.kernel,
        mesh=mesh,
        compiler_params=pltpu.CompilerParams(needs_layout_passes=False),
        out_type=[
            jax.ShapeDtypeStruct((NC, N, D), jnp.float32),
            jax.ShapeDtypeStruct((NW, N), jnp.float32),
        ],
        scratch_types=[
            pltpu.VMEM((2, K), jnp.int32),
            pltpu.VMEM((2, K), jnp.int32),
            pltpu.VMEM((K,), jnp.int32),
            pltpu.VMEM((K,), jnp.int32),
            pltpu.VMEM((N,), jnp.int32),
            pltpu.VMEM((N,), jnp.float32),
            pltpu.VMEM((K,), jnp.float32),
            pltpu.VMEM((K, D), jnp.float32),
            pltpu.VMEM((K, D), jnp.float32),
            pltpu.VMEM_SHARED((N, D), jnp.float32),
            pltpu.SemaphoreType.DMA,
            pltpu.SemaphoreType.DMA,
            pltpu.SemaphoreType.DMA,
            pltpu.SemaphoreType.DMA,
            pltpu.SemaphoreType.DMA,
            pltpu.SemaphoreType.DMA,
        ],
    )(_sc_body)
    outp, sp = sc(h, ei3, api)

    bias2 = bias.reshape(1, D)
    out = pl.pallas_call(
        _combine_body,
        grid=grid,
        in_specs=[
            pl.BlockSpec((bn, D), lambda i: (i, 0)),
            pl.BlockSpec((bn, D), lambda i: (i, 0)),
            pl.BlockSpec((bn, D), lambda i: (i, 0)),
            pl.BlockSpec((bn, NW), lambda i: (i, 0)),
            pl.BlockSpec((bn, 2), lambda i: (i, 0)),
            pl.BlockSpec((1, D), lambda i: (0, 0)),
        ],
        out_specs=pl.BlockSpec((bn, D), lambda i: (i, 0)),
        out_shape=jax.ShapeDtypeStruct((N, D), jnp.float32),
    )(outp[0], outp[1], h, sp.T, a2, bias2)
    return out


# speedup vs baseline: 1.0270x; 1.0270x over previous
import functools

import jax
import jax.numpy as jnp
from jax import lax
from jax.experimental import pallas as pl
from jax.experimental.pallas import tpu as pltpu
from jax.experimental.pallas import tpu_sc as plsc

N = 10000
D = 128
E = 320000

NC = 2
NS = 16
NW = NC * NS
EPW = E // NW
K = 80
NB = EPW // K
ROWS_PT = (N // NS) // 8 * 8
TAIL = N - NS * ROWS_PT
QS = 2048.0


def _proj_body(x_ref, w_ref, as_ref, ad_ref, h_ref, a2_ref, api_ref):
    h = jnp.dot(x_ref[...], w_ref[...], preferred_element_type=jnp.float32)
    h_ref[...] = h
    att2 = jnp.concatenate([as_ref[...], ad_ref[...]], axis=1)
    a2 = jnp.dot(h, att2, preferred_element_type=jnp.float32)
    a2_ref[...] = a2
    aq = jnp.clip(jnp.round(a2 * QS), -32767.0, 32767.0).astype(jnp.int32)
    api_ref[...] = (aq[:, 0:1] << 16) | (aq[:, 1:2] & 0xFFFF)


def _combine_body(p0_ref, p1_ref, h_ref, spt_ref, a2_ref, b_ref, o_ref):
    asum = a2_ref[:, 0] + a2_ref[:, 1]
    wself = jnp.exp(jnp.maximum(asum, 0.2 * asum))
    s = jnp.sum(spt_ref[...], axis=1) + wself + 1e-16
    num = p0_ref[...] + p1_ref[...] + h_ref[...] * wself[:, None]
    o_ref[...] = num / s[:, None] + b_ref[...]


def _sc_body(h_hbm, ei_hbm, api_hbm,
             outp_hbm, sp_hbm,
             sb0, sb1, db0, db1, dst0, dst1, api_v, s_v, w_v, rows0, rows1,
             acc_sh, gsem0, gsem1, ssem0, ssem1, esem0, esem1):
    cid = lax.axis_index("c")
    sid = lax.axis_index("s")
    wid = cid * NS + sid
    eoff = wid * EPW

    pltpu.sync_copy(api_hbm, api_v)

    zeros = jnp.zeros((16,), jnp.float32)

    @plsc.parallel_loop(0, N // 16, unroll=8)
    def _z_s(i):
        s_v[pl.ds(i * 16, 16)] = zeros

    @plsc.parallel_loop(0, K, unroll=4)
    def _z_rows(e):
        for j in range(D // 16):
            rows0[e, pl.ds(j * 16, 16)] = zeros

    base = sid * ROWS_PT
    for r in range(0, ROWS_PT, K):
        n = min(K, ROWS_PT - r)
        pltpu.async_copy(rows0.at[pl.ds(0, n)], acc_sh.at[pl.ds(base + r, n)],
                         gsem0)
    for r in range(0, ROWS_PT, K):
        n = min(K, ROWS_PT - r)
        pltpu.make_async_copy(rows0.at[pl.ds(0, n)],
                              acc_sh.at[pl.ds(base + r, n)], gsem0).wait()

    @pl.when(sid == 0)
    def _zero_tail():
        pltpu.sync_copy(rows0.at[pl.ds(0, TAIL)],
                        acc_sh.at[pl.ds(NS * ROWS_PT, TAIL)])
    plsc.subcore_barrier()

    def _weights(sb, db):
        for j in range(K // 16):
            sidx = sb[pl.ds(j * 16, 16)]
            didx = db[pl.ds(j * 16, 16)]
            sw = plsc.load_gather(api_v, [sidx])
            dw = plsc.load_gather(api_v, [didx])
            av = (sw >> 16).astype(jnp.float32) * (1.0 / QS)
            bv = ((dw << 16) >> 16).astype(jnp.float32) * (1.0 / QS)
            x = av + bv
            wv = jnp.exp(jnp.maximum(x, 0.2 * x))
            w_v[pl.ds(j * 16, 16)] = wv
            plsc.addupdate_scatter(s_v, [didx], wv)

    def _scale_rows(rows):
        @plsc.parallel_loop(0, K, unroll=4)
        def _scale(e):
            wb = plsc.load_gather(w_v, [jnp.full((16,), 0, jnp.int32) + e])
            for j in range(D // 16):
                sl = pl.ds(j * 16, 16)
                rows[e, sl] = rows[e, sl] * wb

    def _snap_dst(db, dst):
        for j in range(K // 16):
            dst[pl.ds(j * 16, 16)] = db[pl.ds(j * 16, 16)]

    def _eb_start(b, sb, db, esem):
        pltpu.async_copy(ei_hbm.at[pl.ds(eoff + b * K, K)], sb, esem)
        pltpu.async_copy(ei_hbm.at[pl.ds(E + eoff + b * K, K)], db, esem)

    def _eb_wait(b, sb, db, esem):
        pltpu.make_async_copy(ei_hbm.at[pl.ds(eoff + b * K, K)], sb,
                              esem).wait()
        pltpu.make_async_copy(ei_hbm.at[pl.ds(E + eoff + b * K, K)], db,
                              esem).wait()

    _eb_start(0, sb0, db0, esem0)
    _eb_start(1, sb1, db1, esem1)
    _eb_wait(0, sb0, db0, esem0)
    pltpu.async_copy(h_hbm.at[sb0], rows0, gsem0)

    def _half(b, sb_c, db_c, rows_c, dst_c, gsem_c, ssem_c, esem_c,
              sb_o, db_o, rows_o, dst_o, gsem_o, ssem_o, esem_o,
              prev_pred, eb_pred):
        _weights(sb_c, db_c)
        _snap_dst(db_c, dst_c)
        if prev_pred is None:
            pltpu.make_async_copy(rows_o, acc_sh.at[dst_o], ssem_o).wait()
        else:
            @pl.when(prev_pred)
            def _wait_prev():
                pltpu.make_async_copy(rows_o, acc_sh.at[dst_o], ssem_o).wait()
        _eb_wait(b + 1, sb_o, db_o, esem_o)
        pltpu.async_copy(h_hbm.at[sb_o], rows_o, gsem_o)
        if eb_pred is None:
            _eb_start(b + 2, sb_c, db_c, esem_c)
        else:
            @pl.when(eb_pred)
            def _next_eb():
                _eb_start(b + 2, sb_c, db_c, esem_c)
        pltpu.make_async_copy(h_hbm.at[sb_c], rows_c, gsem_c).wait()
        _scale_rows(rows_c)
        pltpu.async_copy(rows_c, acc_sh.at[dst_c], ssem_c, add=True)

    def _pair(g, carry):
        b0 = 2 * g
        _half(b0, sb0, db0, rows0, dst0, gsem0, ssem0, esem0,
              sb1, db1, rows1, dst1, gsem1, ssem1, esem1, g >= 1, None)
        _half(b0 + 1, sb1, db1, rows1, dst1, gsem1, ssem1, esem1,
              sb0, db0, rows0, dst0, gsem0, ssem0, esem0, None,
              g < (NB - 1) // 2 - 1)
        return carry

    lax.fori_loop(0, (NB - 1) // 2, _pair, 0)

    _weights(sb0, db0)
    _snap_dst(db0, dst0)
    pltpu.make_async_copy(rows1, acc_sh.at[dst1], ssem1).wait()
    pltpu.make_async_copy(h_hbm.at[sb0], rows0, gsem0).wait()
    _scale_rows(rows0)
    pltpu.async_copy(rows0, acc_sh.at[dst0], ssem0, add=True)
    pltpu.make_async_copy(rows0, acc_sh.at[dst0], ssem0).wait()

    pltpu.sync_copy(s_v, sp_hbm.at[wid])

    plsc.subcore_barrier()
    pltpu.sync_copy(acc_sh.at[pl.ds(base, ROWS_PT)],
                    outp_hbm.at[cid, pl.ds(base, ROWS_PT)])

    @pl.when(sid == 0)
    def _copy_tail():
        pltpu.sync_copy(acc_sh.at[pl.ds(NS * ROWS_PT, TAIL)],
                        outp_hbm.at[cid, pl.ds(NS * ROWS_PT, TAIL)])


def kernel(X, edge_index, W, att_src, att_dst, bias):
    grid = (10,)
    bn = N // grid[0]
    h, a2, api = pl.pallas_call(
        _proj_body,
        grid=grid,
        in_specs=[
            pl.BlockSpec((bn, D), lambda i: (i, 0)),
            pl.BlockSpec((D, D), lambda i: (0, 0)),
            pl.BlockSpec((D, 1), lambda i: (0, 0)),
            pl.BlockSpec((D, 1), lambda i: (0, 0)),
        ],
        out_specs=[
            pl.BlockSpec((bn, D), lambda i: (i, 0)),
            pl.BlockSpec((bn, 2), lambda i: (i, 0)),
            pl.BlockSpec((bn, 1), lambda i: (i, 0)),
        ],
        out_shape=[
            jax.ShapeDtypeStruct((N, D), jnp.float32),
            jax.ShapeDtypeStruct((N, 2), jnp.float32),
            jax.ShapeDtypeStruct((N, 1), jnp.int32),
        ],
    )(X, W, att_src.reshape(D, 1), att_dst.reshape(D, 1))

    mesh = plsc.VectorSubcoreMesh(core_axis_name="c", subcore_axis_name="s")
    sc = functools.partial(
        pl.kernel,
        mesh=mesh,
        compiler_params=pltpu.CompilerParams(needs_layout_passes=False),
        out_type=[
            jax.ShapeDtypeStruct((NC, N, D), jnp.float32),
            jax.ShapeDtypeStruct((NW, N), jnp.float32),
        ],
        scratch_types=[
            pltpu.VMEM((K,), jnp.int32),
            pltpu.VMEM((K,), jnp.int32),
            pltpu.VMEM((K,), jnp.int32),
            pltpu.VMEM((K,), jnp.int32),
            pltpu.VMEM((K,), jnp.int32),
            pltpu.VMEM((K,), jnp.int32),
            pltpu.VMEM((N,), jnp.int32),
            pltpu.VMEM((N,), jnp.float32),
            pltpu.VMEM((K,), jnp.float32),
            pltpu.VMEM((K, D), jnp.float32),
            pltpu.VMEM((K, D), jnp.float32),
            pltpu.VMEM_SHARED((N, D), jnp.float32),
            pltpu.SemaphoreType.DMA,
            pltpu.SemaphoreType.DMA,
            pltpu.SemaphoreType.DMA,
            pltpu.SemaphoreType.DMA,
            pltpu.SemaphoreType.DMA,
            pltpu.SemaphoreType.DMA,
        ],
    )(_sc_body)
    outp, sp = sc(h, edge_index.reshape(2 * E), api.reshape(N))

    bias2 = bias.reshape(1, D)
    out = pl.pallas_call(
        _combine_body,
        grid=grid,
        in_specs=[
            pl.BlockSpec((bn, D), lambda i: (i, 0)),
            pl.BlockSpec((bn, D), lambda i: (i, 0)),
            pl.BlockSpec((bn, D), lambda i: (i, 0)),
            pl.BlockSpec((bn, NW), lambda i: (i, 0)),
            pl.BlockSpec((bn, 2), lambda i: (i, 0)),
            pl.BlockSpec((1, D), lambda i: (0, 0)),
        ],
        out_specs=pl.BlockSpec((bn, D), lambda i: (i, 0)),
        out_shape=jax.ShapeDtypeStruct((N, D), jnp.float32),
    )(outp[0], outp[1], h, sp.T, a2, bias2)
    return out


# speedup vs baseline: 1.0458x; 1.0183x over previous
import functools

import jax
import jax.numpy as jnp
from jax import lax
from jax.experimental import pallas as pl
from jax.experimental.pallas import tpu as pltpu
from jax.experimental.pallas import tpu_sc as plsc

N = 10000
D = 128
E = 320000

NC = 2
NS = 16
NW = NC * NS
EPW = E // NW
K = 80
NB = EPW // K
ROWS_PT = (N // NS) // 8 * 8
TAIL = N - NS * ROWS_PT
QS = 2048.0


def _proj_body(x_ref, w_ref, as_ref, ad_ref, h_ref, a2_ref, api_ref):
    h = jnp.dot(x_ref[...], w_ref[...], preferred_element_type=jnp.float32)
    h_ref[...] = h
    att2 = jnp.concatenate([as_ref[...], ad_ref[...]], axis=1)
    a2 = jnp.dot(h, att2, preferred_element_type=jnp.float32)
    a2_ref[...] = a2
    aq = jnp.clip(jnp.round(a2 * QS), -32767.0, 32767.0).astype(jnp.int32)
    api_ref[...] = (aq[:, 0:1] << 16) | (aq[:, 1:2] & 0xFFFF)


def _combine_body(p0_ref, p1_ref, h_ref, se_ref, a2_ref, b_ref, o_ref):
    asum = a2_ref[:, 0] + a2_ref[:, 1]
    wself = jnp.exp(jnp.maximum(asum, 0.2 * asum))
    s = se_ref[:, 0] + wself + 1e-16
    num = p0_ref[...] + p1_ref[...] + h_ref[...] * wself[:, None]
    o_ref[...] = num / s[:, None] + b_ref[...]


def _sc_body(h_hbm, ei_hbm, api_hbm,
             outp_hbm, sp_hbm,
             sb0, sb1, db0, db1, dst0, dst1, api_v, s_v, w_v, rows0, rows1,
             acc_sh, gsem0, gsem1, ssem0, ssem1, esem0, esem1):
    cid = lax.axis_index("c")
    sid = lax.axis_index("s")
    wid = cid * NS + sid
    eoff = wid * EPW

    pltpu.sync_copy(api_hbm, api_v)

    zeros = jnp.zeros((16,), jnp.float32)

    @plsc.parallel_loop(0, N // 16, unroll=8)
    def _z_s(i):
        s_v[pl.ds(i * 16, 16)] = zeros

    @plsc.parallel_loop(0, K, unroll=4)
    def _z_rows(e):
        for j in range(D // 16):
            rows0[e, pl.ds(j * 16, 16)] = zeros

    base = sid * ROWS_PT
    for r in range(0, ROWS_PT, K):
        n = min(K, ROWS_PT - r)
        pltpu.async_copy(rows0.at[pl.ds(0, n)], acc_sh.at[pl.ds(base + r, n)],
                         gsem0)
    for r in range(0, ROWS_PT, K):
        n = min(K, ROWS_PT - r)
        pltpu.make_async_copy(rows0.at[pl.ds(0, n)],
                              acc_sh.at[pl.ds(base + r, n)], gsem0).wait()

    @pl.when(sid == 0)
    def _zero_tail():
        pltpu.sync_copy(rows0.at[pl.ds(0, TAIL)],
                        acc_sh.at[pl.ds(NS * ROWS_PT, TAIL)])
    plsc.subcore_barrier()

    def _weights(sb, db):
        for j in range(K // 16):
            sidx = sb[pl.ds(j * 16, 16)]
            didx = db[pl.ds(j * 16, 16)]
            sw = plsc.load_gather(api_v, [sidx])
            dw = plsc.load_gather(api_v, [didx])
            av = (sw >> 16).astype(jnp.float32) * (1.0 / QS)
            bv = ((dw << 16) >> 16).astype(jnp.float32) * (1.0 / QS)
            x = av + bv
            wv = jnp.exp(jnp.maximum(x, 0.2 * x))
            w_v[pl.ds(j * 16, 16)] = wv
            plsc.addupdate_scatter(s_v, [didx], wv)

    def _scale_rows(rows):
        @plsc.parallel_loop(0, K, unroll=4)
        def _scale(e):
            wb = plsc.load_gather(w_v, [jnp.full((16,), 0, jnp.int32) + e])
            for j in range(D // 16):
                sl = pl.ds(j * 16, 16)
                rows[e, sl] = rows[e, sl] * wb

    def _snap_dst(db, dst):
        for j in range(K // 16):
            dst[pl.ds(j * 16, 16)] = db[pl.ds(j * 16, 16)]

    def _eb_start(b, sb, db, esem):
        pltpu.async_copy(ei_hbm.at[pl.ds(eoff + b * K, K)], sb, esem)
        pltpu.async_copy(ei_hbm.at[pl.ds(E + eoff + b * K, K)], db, esem)

    def _eb_wait(b, sb, db, esem):
        pltpu.make_async_copy(ei_hbm.at[pl.ds(eoff + b * K, K)], sb,
                              esem).wait()
        pltpu.make_async_copy(ei_hbm.at[pl.ds(E + eoff + b * K, K)], db,
                              esem).wait()

    _eb_start(0, sb0, db0, esem0)
    _eb_start(1, sb1, db1, esem1)
    _eb_wait(0, sb0, db0, esem0)
    pltpu.async_copy(h_hbm.at[sb0], rows0, gsem0)

    def _half(b, sb_c, db_c, rows_c, dst_c, gsem_c, ssem_c, esem_c,
              sb_o, db_o, rows_o, dst_o, gsem_o, ssem_o, esem_o,
              prev_pred, eb_pred):
        _weights(sb_c, db_c)
        _snap_dst(db_c, dst_c)
        if prev_pred is None:
            pltpu.make_async_copy(rows_o, acc_sh.at[dst_o], ssem_o).wait()
        else:
            @pl.when(prev_pred)
            def _wait_prev():
                pltpu.make_async_copy(rows_o, acc_sh.at[dst_o], ssem_o).wait()
        _eb_wait(b + 1, sb_o, db_o, esem_o)
        pltpu.async_copy(h_hbm.at[sb_o], rows_o, gsem_o)
        if eb_pred is None:
            _eb_start(b + 2, sb_c, db_c, esem_c)
        else:
            @pl.when(eb_pred)
            def _next_eb():
                _eb_start(b + 2, sb_c, db_c, esem_c)
        pltpu.make_async_copy(h_hbm.at[sb_c], rows_c, gsem_c).wait()
        _scale_rows(rows_c)
        pltpu.async_copy(rows_c, acc_sh.at[dst_c], ssem_c, add=True)

    def _pair(g, carry):
        b0 = 2 * g
        _half(b0, sb0, db0, rows0, dst0, gsem0, ssem0, esem0,
              sb1, db1, rows1, dst1, gsem1, ssem1, esem1, g >= 1, None)
        _half(b0 + 1, sb1, db1, rows1, dst1, gsem1, ssem1, esem1,
              sb0, db0, rows0, dst0, gsem0, ssem0, esem0, None,
              g < (NB - 1) // 2 - 1)
        return carry

    lax.fori_loop(0, (NB - 1) // 2, _pair, 0)

    _weights(sb0, db0)
    _snap_dst(db0, dst0)
    pltpu.make_async_copy(rows1, acc_sh.at[dst1], ssem1).wait()
    pltpu.make_async_copy(h_hbm.at[sb0], rows0, gsem0).wait()
    _scale_rows(rows0)
    pltpu.async_copy(rows0, acc_sh.at[dst0], ssem0, add=True)
    pltpu.make_async_copy(rows0, acc_sh.at[dst0], ssem0).wait()

    pltpu.sync_copy(s_v, sp_hbm.at[wid])

    plsc.subcore_barrier()
    pltpu.sync_copy(acc_sh.at[pl.ds(base, ROWS_PT)],
                    outp_hbm.at[cid, pl.ds(base, ROWS_PT)])

    @pl.when(sid == 0)
    def _copy_tail():
        pltpu.sync_copy(acc_sh.at[pl.ds(NS * ROWS_PT, TAIL)],
                        outp_hbm.at[cid, pl.ds(NS * ROWS_PT, TAIL)])


def kernel(X, edge_index, W, att_src, att_dst, bias):
    grid = (5,)
    bn = N // grid[0]
    h, a2, api = pl.pallas_call(
        _proj_body,
        grid=grid,
        in_specs=[
            pl.BlockSpec((bn, D), lambda i: (i, 0)),
            pl.BlockSpec((D, D), lambda i: (0, 0)),
            pl.BlockSpec((D, 1), lambda i: (0, 0)),
            pl.BlockSpec((D, 1), lambda i: (0, 0)),
        ],
        out_specs=[
            pl.BlockSpec((bn, D), lambda i: (i, 0)),
            pl.BlockSpec((bn, 2), lambda i: (i, 0)),
            pl.BlockSpec((bn, 1), lambda i: (i, 0)),
        ],
        out_shape=[
            jax.ShapeDtypeStruct((N, D), jnp.float32),
            jax.ShapeDtypeStruct((N, 2), jnp.float32),
            jax.ShapeDtypeStruct((N, 1), jnp.int32),
        ],
    )(X, W, att_src.reshape(D, 1), att_dst.reshape(D, 1))

    mesh = plsc.VectorSubcoreMesh(core_axis_name="c", subcore_axis_name="s")
    sc = functools.partial(
        pl.kernel,
        mesh=mesh,
        compiler_params=pltpu.CompilerParams(needs_layout_passes=False),
        out_type=[
            jax.ShapeDtypeStruct((NC, N, D), jnp.float32),
            jax.ShapeDtypeStruct((NW, N), jnp.float32),
        ],
        scratch_types=[
            pltpu.VMEM((K,), jnp.int32),
            pltpu.VMEM((K,), jnp.int32),
            pltpu.VMEM((K,), jnp.int32),
            pltpu.VMEM((K,), jnp.int32),
            pltpu.VMEM((K,), jnp.int32),
            pltpu.VMEM((K,), jnp.int32),
            pltpu.VMEM((N,), jnp.int32),
            pltpu.VMEM((N,), jnp.float32),
            pltpu.VMEM((K,), jnp.float32),
            pltpu.VMEM((K, D), jnp.float32),
            pltpu.VMEM((K, D), jnp.float32),
            pltpu.VMEM_SHARED((N, D), jnp.float32),
            pltpu.SemaphoreType.DMA,
            pltpu.SemaphoreType.DMA,
            pltpu.SemaphoreType.DMA,
            pltpu.SemaphoreType.DMA,
            pltpu.SemaphoreType.DMA,
            pltpu.SemaphoreType.DMA,
        ],
    )(_sc_body)
    outp, sp = sc(h, edge_index.reshape(2 * E), api.reshape(N))

    bias2 = bias.reshape(1, D)
    s_e = jnp.sum(sp, axis=0).reshape(N, 1)
    out = pl.pallas_call(
        _combine_body,
        grid=grid,
        in_specs=[
            pl.BlockSpec((bn, D), lambda i: (i, 0)),
            pl.BlockSpec((bn, D), lambda i: (i, 0)),
            pl.BlockSpec((bn, D), lambda i: (i, 0)),
            pl.BlockSpec((bn, 1), lambda i: (i, 0)),
            pl.BlockSpec((bn, 2), lambda i: (i, 0)),
            pl.BlockSpec((1, D), lambda i: (0, 0)),
        ],
        out_specs=pl.BlockSpec((bn, D), lambda i: (i, 0)),
        out_shape=jax.ShapeDtypeStruct((N, D), jnp.float32),
    )(outp[0], outp[1], h, s_e, a2, bias2)
    return out


# speedup vs baseline: 1.0729x; 1.0259x over previous
import functools

import jax
import jax.numpy as jnp
from jax import lax
from jax.experimental import pallas as pl
from jax.experimental.pallas import tpu as pltpu
from jax.experimental.pallas import tpu_sc as plsc

N = 10000
D = 128
E = 320000

NC = 2
NS = 16
NW = NC * NS
EPW = E // NW
K = 80
NB = EPW // K
ROWS_PT = (N // NS) // 8 * 8
TAIL = N - NS * ROWS_PT
QS = 2048.0


def _proj_body(x_ref, w_ref, as_ref, ad_ref, h_ref, a2_ref, api_ref):
    h = jnp.dot(x_ref[...], w_ref[...], preferred_element_type=jnp.float32)
    h_ref[...] = h
    att2 = jnp.concatenate([as_ref[...], ad_ref[...]], axis=1)
    a2 = jnp.dot(h, att2, preferred_element_type=jnp.float32)
    a2_ref[...] = a2
    aq = jnp.clip(jnp.round(a2 * QS), -32767.0, 32767.0).astype(jnp.int32)
    api_ref[...] = (aq[:, 0:1] << 16) | (aq[:, 1:2] & 0xFFFF)


def _combine_body(p0_ref, p1_ref, h_ref, se_ref, a2_ref, b_ref, o_ref):
    asum = a2_ref[:, 0] + a2_ref[:, 1]
    wself = jnp.exp(jnp.maximum(asum, 0.2 * asum))
    s = se_ref[:, 0] + wself + 1e-16
    num = p0_ref[...] + p1_ref[...] + h_ref[...] * wself[:, None]
    o_ref[...] = num / s[:, None] + b_ref[...]


def _sc_body(h_hbm, ei_hbm, api_hbm,
             outp_hbm, sp_hbm,
             sb0, sb1, db0, db1, dst0, dst1, api_v, s_v, w_v, rows0, rows1,
             acc_sh, gsa0, gsb0, gsa1, gsb1, ssem0, ssem1, esem0, esem1):
    cid = lax.axis_index("c")
    sid = lax.axis_index("s")
    wid = cid * NS + sid
    eoff = wid * EPW

    pltpu.sync_copy(api_hbm, api_v)

    zeros = jnp.zeros((16,), jnp.float32)

    @plsc.parallel_loop(0, N // 16, unroll=8)
    def _z_s(i):
        s_v[pl.ds(i * 16, 16)] = zeros

    @plsc.parallel_loop(0, K, unroll=4)
    def _z_rows(e):
        for j in range(D // 16):
            rows0[e, pl.ds(j * 16, 16)] = zeros

    base = sid * ROWS_PT
    for r in range(0, ROWS_PT, K):
        n = min(K, ROWS_PT - r)
        pltpu.async_copy(rows0.at[pl.ds(0, n)], acc_sh.at[pl.ds(base + r, n)],
                         gsa0)
    for r in range(0, ROWS_PT, K):
        n = min(K, ROWS_PT - r)
        pltpu.make_async_copy(rows0.at[pl.ds(0, n)],
                              acc_sh.at[pl.ds(base + r, n)], gsa0).wait()

    @pl.when(sid == 0)
    def _zero_tail():
        pltpu.sync_copy(rows0.at[pl.ds(0, TAIL)],
                        acc_sh.at[pl.ds(NS * ROWS_PT, TAIL)])
    plsc.subcore_barrier()

    def _weights(sb, db):
        for j in range(K // 16):
            sidx = sb[pl.ds(j * 16, 16)]
            didx = db[pl.ds(j * 16, 16)]
            sw = plsc.load_gather(api_v, [sidx])
            dw = plsc.load_gather(api_v, [didx])
            av = (sw >> 16).astype(jnp.float32) * (1.0 / QS)
            bv = ((dw << 16) >> 16).astype(jnp.float32) * (1.0 / QS)
            x = av + bv
            wv = jnp.exp(jnp.maximum(x, 0.2 * x))
            w_v[pl.ds(j * 16, 16)] = wv
            plsc.addupdate_scatter(s_v, [didx], wv)

    def _scale_half(rows, lo):
        @plsc.parallel_loop(lo, lo + K // 2, unroll=4)
        def _scale(e):
            wb = plsc.load_gather(w_v, [jnp.full((16,), 0, jnp.int32) + e])
            for j in range(D // 16):
                sl = pl.ds(j * 16, 16)
                rows[e, sl] = rows[e, sl] * wb

    def _gather_start(sb, rows, gsa, gsb):
        pltpu.async_copy(h_hbm.at[sb.at[pl.ds(0, K // 2)]],
                         rows.at[pl.ds(0, K // 2)], gsa)
        pltpu.async_copy(h_hbm.at[sb.at[pl.ds(K // 2, K // 2)]],
                         rows.at[pl.ds(K // 2, K // 2)], gsb)

    def _gather_wait_a(sb, rows, gsa):
        pltpu.make_async_copy(h_hbm.at[sb.at[pl.ds(0, K // 2)]],
                              rows.at[pl.ds(0, K // 2)], gsa).wait()

    def _gather_wait_b(sb, rows, gsb):
        pltpu.make_async_copy(h_hbm.at[sb.at[pl.ds(K // 2, K // 2)]],
                              rows.at[pl.ds(K // 2, K // 2)], gsb).wait()

    def _snap_dst(db, dst):
        for j in range(K // 16):
            dst[pl.ds(j * 16, 16)] = db[pl.ds(j * 16, 16)]

    def _eb_start(b, sb, db, esem):
        pltpu.async_copy(ei_hbm.at[pl.ds(eoff + b * K, K)], sb, esem)
        pltpu.async_copy(ei_hbm.at[pl.ds(E + eoff + b * K, K)], db, esem)

    def _eb_wait(b, sb, db, esem):
        pltpu.make_async_copy(ei_hbm.at[pl.ds(eoff + b * K, K)], sb,
                              esem).wait()
        pltpu.make_async_copy(ei_hbm.at[pl.ds(E + eoff + b * K, K)], db,
                              esem).wait()

    _eb_start(0, sb0, db0, esem0)
    _eb_start(1, sb1, db1, esem1)
    _eb_wait(0, sb0, db0, esem0)
    _gather_start(sb0, rows0, gsa0, gsb0)

    def _half(b, sb_c, db_c, rows_c, dst_c, gsa_c, gsb_c, ssem_c, esem_c,
              sb_o, db_o, rows_o, dst_o, gsa_o, gsb_o, ssem_o, esem_o,
              prev_pred, eb_pred):
        _weights(sb_c, db_c)
        _snap_dst(db_c, dst_c)
        if prev_pred is None:
            pltpu.make_async_copy(rows_o, acc_sh.at[dst_o], ssem_o).wait()
        else:
            @pl.when(prev_pred)
            def _wait_prev():
                pltpu.make_async_copy(rows_o, acc_sh.at[dst_o], ssem_o).wait()
        _eb_wait(b + 1, sb_o, db_o, esem_o)
        _gather_start(sb_o, rows_o, gsa_o, gsb_o)
        if eb_pred is None:
            _eb_start(b + 2, sb_c, db_c, esem_c)
        else:
            @pl.when(eb_pred)
            def _next_eb():
                _eb_start(b + 2, sb_c, db_c, esem_c)
        _gather_wait_a(sb_c, rows_c, gsa_c)
        _scale_half(rows_c, 0)
        _gather_wait_b(sb_c, rows_c, gsb_c)
        _scale_half(rows_c, K // 2)
        pltpu.async_copy(rows_c, acc_sh.at[dst_c], ssem_c, add=True)

    def _pair(g, carry):
        b0 = 2 * g
        _half(b0, sb0, db0, rows0, dst0, gsa0, gsb0, ssem0, esem0,
              sb1, db1, rows1, dst1, gsa1, gsb1, ssem1, esem1, g >= 1, None)
        _half(b0 + 1, sb1, db1, rows1, dst1, gsa1, gsb1, ssem1, esem1,
              sb0, db0, rows0, dst0, gsa0, gsb0, ssem0, esem0, None,
              g < (NB - 1) // 2 - 1)
        return carry

    lax.fori_loop(0, (NB - 1) // 2, _pair, 0)

    _weights(sb0, db0)
    _snap_dst(db0, dst0)
    pltpu.make_async_copy(rows1, acc_sh.at[dst1], ssem1).wait()
    _gather_wait_a(sb0, rows0, gsa0)
    _scale_half(rows0, 0)
    _gather_wait_b(sb0, rows0, gsb0)
    _scale_half(rows0, K // 2)
    pltpu.async_copy(rows0, acc_sh.at[dst0], ssem0, add=True)
    pltpu.make_async_copy(rows0, acc_sh.at[dst0], ssem0).wait()

    pltpu.sync_copy(s_v, sp_hbm.at[wid])

    plsc.subcore_barrier()
    pltpu.sync_copy(acc_sh.at[pl.ds(base, ROWS_PT)],
                    outp_hbm.at[cid, pl.ds(base, ROWS_PT)])

    @pl.when(sid == 0)
    def _copy_tail():
        pltpu.sync_copy(acc_sh.at[pl.ds(NS * ROWS_PT, TAIL)],
                        outp_hbm.at[cid, pl.ds(NS * ROWS_PT, TAIL)])


def kernel(X, edge_index, W, att_src, att_dst, bias):
    grid = (5,)
    bn = N // grid[0]
    h, a2, api = pl.pallas_call(
        _proj_body,
        grid=grid,
        in_specs=[
            pl.BlockSpec((bn, D), lambda i: (i, 0)),
            pl.BlockSpec((D, D), lambda i: (0, 0)),
            pl.BlockSpec((D, 1), lambda i: (0, 0)),
            pl.BlockSpec((D, 1), lambda i: (0, 0)),
        ],
        out_specs=[
            pl.BlockSpec((bn, D), lambda i: (i, 0)),
            pl.BlockSpec((bn, 2), lambda i: (i, 0)),
            pl.BlockSpec((bn, 1), lambda i: (i, 0)),
        ],
        out_shape=[
            jax.ShapeDtypeStruct((N, D), jnp.float32),
            jax.ShapeDtypeStruct((N, 2), jnp.float32),
            jax.ShapeDtypeStruct((N, 1), jnp.int32),
        ],
    )(X, W, att_src.reshape(D, 1), att_dst.reshape(D, 1))

    mesh = plsc.VectorSubcoreMesh(core_axis_name="c", subcore_axis_name="s")
    sc = functools.partial(
        pl.kernel,
        mesh=mesh,
        compiler_params=pltpu.CompilerParams(needs_layout_passes=False),
        out_type=[
            jax.ShapeDtypeStruct((NC, N, D), jnp.float32),
            jax.ShapeDtypeStruct((NW, N), jnp.float32),
        ],
        scratch_types=[
            pltpu.VMEM((K,), jnp.int32),
            pltpu.VMEM((K,), jnp.int32),
            pltpu.VMEM((K,), jnp.int32),
            pltpu.VMEM((K,), jnp.int32),
            pltpu.VMEM((K,), jnp.int32),
            pltpu.VMEM((K,), jnp.int32),
            pltpu.VMEM((N,), jnp.int32),
            pltpu.VMEM((N,), jnp.float32),
            pltpu.VMEM((K,), jnp.float32),
            pltpu.VMEM((K, D), jnp.float32),
            pltpu.VMEM((K, D), jnp.float32),
            pltpu.VMEM_SHARED((N, D), jnp.float32),
            pltpu.SemaphoreType.DMA,
            pltpu.SemaphoreType.DMA,
            pltpu.SemaphoreType.DMA,
            pltpu.SemaphoreType.DMA,
            pltpu.SemaphoreType.DMA,
            pltpu.SemaphoreType.DMA,
            pltpu.SemaphoreType.DMA,
            pltpu.SemaphoreType.DMA,
        ],
    )(_sc_body)
    outp, sp = sc(h, edge_index.reshape(2 * E), api.reshape(N))

    bias2 = bias.reshape(1, D)
    s_e = jnp.sum(sp, axis=0).reshape(N, 1)
    out = pl.pallas_call(
        _combine_body,
        grid=grid,
        in_specs=[
            pl.BlockSpec((bn, D), lambda i: (i, 0)),
            pl.BlockSpec((bn, D), lambda i: (i, 0)),
            pl.BlockSpec((bn, D), lambda i: (i, 0)),
            pl.BlockSpec((bn, 1), lambda i: (i, 0)),
            pl.BlockSpec((bn, 2), lambda i: (i, 0)),
            pl.BlockSpec((1, D), lambda i: (0, 0)),
        ],
        out_specs=pl.BlockSpec((bn, D), lambda i: (i, 0)),
        out_shape=jax.ShapeDtypeStruct((N, D), jnp.float32),
    )(outp[0], outp[1], h, s_e, a2, bias2)
    return out
